# baseline (speedup 1.0000x reference)
import functools

import jax
import jax.numpy as jnp
from jax import lax
from jax.experimental import pallas as pl
from jax.experimental.pallas import tpu as pltpu
from jax.experimental.pallas import tpu_sc as plsc


def kernel(x, pos_table):
    B, S, D = x.shape
    x2 = x.reshape(B * S, D)
    NC, NS = 2, 16
    NW = NC * NS
    SPW = S // NW
    P = 16
    NCHUNK = SPW // P
    CPR = D // 16
    mesh = plsc.VectorSubcoreMesh(
        core_axis_name="c", subcore_axis_name="s", num_cores=NC, num_subcores=NS
    )

    @functools.partial(
        pl.kernel,
        out_type=jax.ShapeDtypeStruct((B * S, D), jnp.float32),
        mesh=mesh,
        scratch_types=[
            pltpu.VMEM((P, D), jnp.float32),
            pltpu.VMEM((B, P, D), jnp.float32),
        ],
    )
    def k(x_hbm, t_hbm, o_hbm, tt, xb):
        wid = lax.axis_index("s") * NC + lax.axis_index("c")
        base = wid * SPW

        def chunk(ci, carry):
            p0 = base + ci * P
            pltpu.sync_copy(t_hbm.at[pl.ds(p0, P)], tt)
            for b in range(B):
                pltpu.sync_copy(x_hbm.at[pl.ds(b * S + p0, P)], xb.at[b])

            def row(r, rc):
                def col(c, cc):
                    sl = pl.ds(c * 16, 16)
                    tv = tt[r, sl]
                    for b in range(B):
                        plsc.addupdate(xb.at[b, r, sl], tv)
                    return cc

                lax.fori_loop(0, CPR, col, 0, unroll=4)
                return rc

            lax.fori_loop(0, P, row, 0)
            for b in range(B):
                pltpu.sync_copy(xb.at[b], o_hbm.at[pl.ds(b * S + p0, P)])
            return carry

        lax.fori_loop(0, NCHUNK, chunk, 0)

    return k(x2, pos_table).reshape(B, S, D)


# speedup vs baseline: 1.7422x; 1.7422x over previous
import functools

import jax
import jax.numpy as jnp
from jax import lax
from jax.experimental import pallas as pl
from jax.experimental.pallas import tpu as pltpu
from jax.experimental.pallas import tpu_sc as plsc

_RING = 3


def kernel(x, pos_table):
    B, S, D = x.shape
    NC, NS = 2, 16
    NW = NC * NS
    SPW = S // NW
    P = 8
    NCHUNK = SPW // P
    CPR = D // 16
    mesh = plsc.VectorSubcoreMesh(
        core_axis_name="c", subcore_axis_name="s", num_cores=NC, num_subcores=NS
    )

    @functools.partial(
        pl.kernel,
        out_type=jax.ShapeDtypeStruct((B, S, D), jnp.float32),
        mesh=mesh,
        scratch_types=[
            pltpu.VMEM((_RING, P, D), jnp.float32),
            pltpu.VMEM((_RING, B, P, D), jnp.float32),
            pltpu.SemaphoreType.DMA((_RING,)),
            pltpu.SemaphoreType.DMA((_RING,)),
        ],
    )
    def k(x_hbm, t_hbm, o_hbm, tt, xb, insem, outsem):
        wid = lax.axis_index("s") * NC + lax.axis_index("c")
        base = wid * SPW

        def in_descs(ci, sl):
            p0 = base + ci * P
            return (
                pltpu.make_async_copy(t_hbm.at[pl.ds(p0, P)], tt.at[sl], insem.at[sl]),
                pltpu.make_async_copy(
                    x_hbm.at[:, pl.ds(p0, P)], xb.at[sl], insem.at[sl]
                ),
            )

        def out_desc(ci, sl):
            p0 = base + ci * P
            return pltpu.make_async_copy(
                xb.at[sl], o_hbm.at[:, pl.ds(p0, P)], outsem.at[sl]
            )

        for d in in_descs(0, 0):
            d.start()
        for ci in range(NCHUNK):
            sl = ci % _RING
            for d in in_descs(ci, sl):
                d.wait()
            if ci >= 2:
                out_desc(ci - 2, (ci + 1) % _RING).wait()
            if ci + 1 < NCHUNK:
                for d in in_descs(ci + 1, (ci + 1) % _RING):
                    d.start()

            def row(r, rc):
                def col(c, cc):
                    slc = pl.ds(c * 16, 16)
                    tv = tt[sl, r, slc]
                    for b in range(B):
                        plsc.addupdate(xb.at[sl, b, r, slc], tv)
                    return cc

                lax.fori_loop(0, CPR, col, 0, unroll=4)
                return rc

            lax.fori_loop(0, P, row, 0)
            out_desc(ci, sl).start()
        for ci in range(NCHUNK - 2, NCHUNK):
            out_desc(ci, ci % _RING).wait()

    return k(x, pos_table)


# speedup vs baseline: 1.7867x; 1.0255x over previous
import functools

import jax
import jax.numpy as jnp
from jax import lax
from jax.experimental import pallas as pl
from jax.experimental.pallas import tpu as pltpu
from jax.experimental.pallas import tpu_sc as plsc

_RING = 3


def kernel(x, pos_table):
    B, S, D = x.shape
    NC, NS = 2, 16
    NW = NC * NS
    SPW = S // NW
    P = 8
    NCHUNK = SPW // P
    CPR = D // 16
    mesh = plsc.VectorSubcoreMesh(
        core_axis_name="c", subcore_axis_name="s", num_cores=NC, num_subcores=NS
    )

    @functools.partial(
        pl.kernel,
        out_type=jax.ShapeDtypeStruct((B, S, D), jnp.float32),
        mesh=mesh,
        scratch_types=[
            pltpu.VMEM((_RING, P, D), jnp.float32),
            pltpu.VMEM((_RING, B, P, D), jnp.float32),
            pltpu.SemaphoreType.DMA((_RING,)),
            pltpu.SemaphoreType.DMA((_RING,)),
        ],
    )
    def k(x_hbm, t_hbm, o_hbm, tt, xb, insem, outsem):
        wid = lax.axis_index("s") * NC + lax.axis_index("c")
        base = wid * SPW

        def in_descs(ci, sl):
            p0 = base + ci * P
            return (
                pltpu.make_async_copy(t_hbm.at[pl.ds(p0, P)], tt.at[sl], insem.at[sl]),
                pltpu.make_async_copy(
                    x_hbm.at[:, pl.ds(p0, P)], xb.at[sl], insem.at[sl]
                ),
            )

        def out_desc(ci, sl):
            p0 = base + ci * P
            return pltpu.make_async_copy(
                xb.at[sl], o_hbm.at[:, pl.ds(p0, P)], outsem.at[sl]
            )

        for d in in_descs(0, 0):
            d.start()
        for ci in range(NCHUNK):
            sl = ci % _RING
            for d in in_descs(ci, sl):
                d.wait()
            if ci >= 2:
                out_desc(ci - 2, (ci + 1) % _RING).wait()
            if ci + 1 < NCHUNK:
                for d in in_descs(ci + 1, (ci + 1) % _RING):
                    d.start()

            def row(r, rc):
                @plsc.parallel_loop(0, D, step=16, unroll=8)
                def _col(j):
                    slc = pl.ds(j, 16)
                    tv = tt[sl, r, slc]
                    for b in range(B):
                        plsc.addupdate(xb.at[sl, b, r, slc], tv)

                return rc

            lax.fori_loop(0, P, row, 0)
            out_desc(ci, sl).start()
        for ci in range(NCHUNK - 2, NCHUNK):
            out_desc(ci, ci % _RING).wait()

    return k(x, pos_table)
